# Initial kernel scaffold; baseline (speedup 1.0000x reference)
import jax
import jax.numpy as jnp
from jax.experimental import pallas as pl


def kernel(index, value, n, weight, bias):
    raise NotImplementedError("write your pallas kernel here")


# speedup vs baseline: 4.5212x; 4.5212x over previous
import functools

import jax
import jax.numpy as jnp
from jax import lax
from jax.experimental import pallas as pl
from jax.experimental.pallas import tpu as pltpu
from jax.experimental.pallas import tpu_sc as plsc

N_ROWS = 10000
NNZ = 320000
OUT_F = 128
LANES = 16
NC = 2
NS = 16
NW = NC * NS
EDGES_PER_W = NNZ // NW
CHUNK = 80
NCHUNK = EDGES_PER_W // CHUNK
GROUPS = OUT_F // LANES
ROWS_PER_TILE = 624
ROWS_REMAINDER = N_ROWS - NS * ROWS_PER_TILE


def _sc_body(row_h, col_h, val_h, w_h, z_h, out_h,
             colbuf, rowbuf, valbuf, rows_v, acc, sem):
    c = lax.axis_index("c")
    s = lax.axis_index("s")
    wid = s * NC + c

    zlo = s * ROWS_PER_TILE
    pltpu.sync_copy(z_h.at[pl.ds(zlo, ROWS_PER_TILE)], acc.at[pl.ds(zlo, ROWS_PER_TILE)])

    @pl.when(s == NS - 1)
    def _zero_tail():
        tail = NS * ROWS_PER_TILE
        pltpu.sync_copy(z_h.at[pl.ds(tail, ROWS_REMAINDER)],
                        acc.at[pl.ds(tail, ROWS_REMAINDER)])

    plsc.subcore_barrier()

    def chunk_body(k, carry):
        base = wid * EDGES_PER_W + k * CHUNK
        pltpu.sync_copy(col_h.at[pl.ds(base, CHUNK)], colbuf)
        pltpu.sync_copy(row_h.at[pl.ds(base, CHUNK)], rowbuf)
        pltpu.sync_copy(val_h.at[pl.ds(base, CHUNK)], valbuf)
        pltpu.async_copy(w_h.at[colbuf], rows_v, sem).wait()

        def group_body(g, carry2):
            val16 = valbuf[pl.ds(g * LANES, LANES)]
            for l in range(LANES):
                vsplat = lax.broadcast(val16[l], (LANES,))
                e = g * LANES + l
                for d in range(GROUPS):
                    sl = pl.ds(d * LANES, LANES)
                    rows_v[e, sl] = rows_v[e, sl] * vsplat
            return carry2

        lax.fori_loop(0, CHUNK // LANES, group_body, 0, unroll=False)
        pltpu.sync_copy(rows_v, acc.at[rowbuf], add=True)
        return carry

    lax.fori_loop(0, NCHUNK, chunk_body, 0, unroll=False)
    plsc.subcore_barrier()
    pltpu.sync_copy(acc.at[pl.ds(zlo, ROWS_PER_TILE)],
                    out_h.at[c, pl.ds(zlo, ROWS_PER_TILE)])

    @pl.when(s == NS - 1)
    def _out_tail():
        tail = NS * ROWS_PER_TILE
        pltpu.sync_copy(acc.at[pl.ds(tail, ROWS_REMAINDER)],
                        out_h.at[c, pl.ds(tail, ROWS_REMAINDER)])


def _combine_body(p_ref, b_ref, o_ref):
    o_ref[...] = p_ref[0] + p_ref[1] + b_ref[...]


@jax.jit
def _run(row, col, value, weight, bias):
    zeros = jnp.zeros((N_ROWS, OUT_F), jnp.float32)
    mesh = plsc.VectorSubcoreMesh(core_axis_name="c", subcore_axis_name="s")
    partials = pl.kernel(
        _sc_body,
        out_type=jax.ShapeDtypeStruct((NC, N_ROWS, OUT_F), jnp.float32),
        mesh=mesh,
        scratch_types=[
            pltpu.VMEM((CHUNK,), jnp.int32),
            pltpu.VMEM((CHUNK,), jnp.int32),
            pltpu.VMEM((CHUNK,), jnp.float32),
            pltpu.VMEM((CHUNK, OUT_F), jnp.float32),
            pltpu.VMEM_SHARED((N_ROWS, OUT_F), jnp.float32),
            pltpu.SemaphoreType.DMA,
        ],
    )(row, col, value, weight, zeros)

    blk = 2000
    out = pl.pallas_call(
        _combine_body,
        grid=(N_ROWS // blk,),
        in_specs=[
            pl.BlockSpec((NC, blk, OUT_F), lambda i: (0, i, 0)),
            pl.BlockSpec((1, OUT_F), lambda i: (0, 0)),
        ],
        out_specs=pl.BlockSpec((blk, OUT_F), lambda i: (i, 0)),
        out_shape=jax.ShapeDtypeStruct((N_ROWS, OUT_F), jnp.float32),
    )(partials, bias.reshape(1, OUT_F))
    return out


def kernel(index, value, n, weight, bias):
    row = index[0].astype(jnp.int32)
    col = index[1].astype(jnp.int32)
    return _run(row, col, value.astype(jnp.float32), weight, bias)
